# Initial kernel scaffold; baseline (speedup 1.0000x reference)
import jax
import jax.numpy as jnp
from jax.experimental import pallas as pl


def kernel(a, b, embedding_table):
    raise NotImplementedError("write your pallas kernel here")


# speedup vs baseline: 3.2732x; 3.2732x over previous
import functools

import jax
import jax.numpy as jnp
from jax import lax
from jax.experimental import pallas as pl
from jax.experimental.pallas import tpu as pltpu
from jax.experimental.pallas import tpu_sc as plsc

D = 128
CHUNK = 128


@functools.lru_cache(maxsize=None)
def _make_gather(total_rows: int):
    info = plsc.get_sparse_core_info()
    nw = info.num_cores * info.num_subcores
    assert total_rows % (nw * CHUNK) == 0
    rows_per_w = total_rows // nw
    nchunks = rows_per_w // CHUNK
    mesh = plsc.VectorSubcoreMesh(core_axis_name="c", subcore_axis_name="s")

    @functools.partial(
        pl.kernel,
        mesh=mesh,
        out_type=jax.ShapeDtypeStruct((total_rows, D), jnp.float32),
        scratch_types=[
            pltpu.VMEM((rows_per_w,), jnp.int32),
            pltpu.VMEM((CHUNK, D), jnp.float32),
            pltpu.SemaphoreType.DMA,
        ],
    )
    def gather_kernel(idx_hbm, table_hbm, out_hbm, idx_v, rows_v, sem):
        wid = lax.axis_index("s") * info.num_cores + lax.axis_index("c")
        base = pl.multiple_of(wid * rows_per_w, 8)
        pltpu.sync_copy(idx_hbm.at[pl.ds(base, rows_per_w)], idx_v)

        def body(j, carry):
            off = pl.multiple_of(j * CHUNK, 8)
            pltpu.async_copy(
                table_hbm.at[idx_v.at[pl.ds(off, CHUNK)]], rows_v, sem
            ).wait()
            pltpu.sync_copy(rows_v, out_hbm.at[pl.ds(base + off, CHUNK)])
            return carry

        lax.fori_loop(0, nchunks, body, 0)

    return gather_kernel


def kernel(a, b, embedding_table):
    batch, hist = a.shape
    idx = jnp.concatenate(
        [a.reshape(-1).astype(jnp.int32), b.reshape(-1).astype(jnp.int32)]
    )
    flat = _make_gather(idx.shape[0])(idx, embedding_table)
    return flat.reshape(2, batch, hist, D)


# speedup vs baseline: 3.6887x; 1.1269x over previous
import functools

import jax
import jax.numpy as jnp
from jax import lax
from jax.experimental import pallas as pl
from jax.experimental.pallas import tpu as pltpu
from jax.experimental.pallas import tpu_sc as plsc

D = 128
CHUNK = 128
NBUF = 4


@functools.lru_cache(maxsize=None)
def _make_gather(total_rows: int):
    info = plsc.get_sparse_core_info()
    nw = info.num_cores * info.num_subcores
    assert total_rows % (nw * CHUNK * NBUF) == 0
    rows_per_w = total_rows // nw
    ngroups = rows_per_w // CHUNK
    nsteps = ngroups // NBUF
    mesh = plsc.VectorSubcoreMesh(core_axis_name="c", subcore_axis_name="s")

    @functools.partial(
        pl.kernel,
        mesh=mesh,
        out_type=jax.ShapeDtypeStruct((total_rows, D), jnp.float32),
        scratch_types=[
            pltpu.VMEM((rows_per_w,), jnp.int32),
            pltpu.VMEM((NBUF, CHUNK, D), jnp.float32),
        ]
        + [pltpu.SemaphoreType.DMA] * NBUF
        + [pltpu.SemaphoreType.DMA] * NBUF,
    )
    def gather_kernel(idx_hbm, table_hbm, out_hbm, idx_v, rows_v, *sems):
        gsem = sems[:NBUF]
        ssem = sems[NBUF:]
        wid = lax.axis_index("s") * info.num_cores + lax.axis_index("c")
        base = pl.multiple_of(wid * rows_per_w, 8)
        pltpu.sync_copy(idx_hbm.at[pl.ds(base, rows_per_w)], idx_v)

        def gather_start(g, b):
            off = pl.multiple_of(g * CHUNK, 8)
            pltpu.async_copy(
                table_hbm.at[idx_v.at[pl.ds(off, CHUNK)]], rows_v.at[b], gsem[b]
            )

        def gather_wait(g, b):
            off = pl.multiple_of(g * CHUNK, 8)
            pltpu.make_async_copy(
                table_hbm.at[idx_v.at[pl.ds(off, CHUNK)]], rows_v.at[b], gsem[b]
            ).wait()

        def store_start(g, b):
            off = pl.multiple_of(g * CHUNK, 8)
            pltpu.async_copy(
                rows_v.at[b], out_hbm.at[pl.ds(base + off, CHUNK)], ssem[b]
            )

        def store_wait(g, b):
            off = pl.multiple_of(g * CHUNK, 8)
            pltpu.make_async_copy(
                rows_v.at[b], out_hbm.at[pl.ds(base + off, CHUNK)], ssem[b]
            ).wait()

        for b in range(NBUF):
            gather_start(b, b)
        for b in range(NBUF):
            gather_wait(b, b)
            store_start(b, b)

        def body(p, carry):
            g0 = p * NBUF
            for b in range(NBUF):
                store_wait(g0 + b - NBUF, b)
                gather_start(g0 + b, b)
            for b in range(NBUF):
                gather_wait(g0 + b, b)
                store_start(g0 + b, b)
            return carry

        lax.fori_loop(1, nsteps, body, 0)

        gl = (nsteps - 1) * NBUF
        for b in range(NBUF):
            store_wait(gl + b, b)

    return gather_kernel


def kernel(a, b, embedding_table):
    batch, hist = a.shape
    idx = jnp.concatenate(
        [a.reshape(-1).astype(jnp.int32), b.reshape(-1).astype(jnp.int32)]
    )
    flat = _make_gather(idx.shape[0])(idx, embedding_table)
    return flat.reshape(2, batch, hist, D)


# speedup vs baseline: 6.3843x; 1.7308x over previous
import functools

import jax
import jax.numpy as jnp
from jax import lax
from jax.experimental import pallas as pl
from jax.experimental.pallas import tpu as pltpu
from jax.experimental.pallas import tpu_sc as plsc

D = 128
HP = 56
NBUF = 4


@functools.lru_cache(maxsize=None)
def _make_gather(nseq: int, batch: int, hist: int):
    info = plsc.get_sparse_core_info()
    nw = info.num_cores * info.num_subcores
    npairs = nseq * batch
    assert npairs % (nw * NBUF) == 0 and batch % (npairs // nw) == 0
    pairs_per_w = npairs // nw
    w_per_seq = batch // pairs_per_w
    nsteps = pairs_per_w // NBUF
    mesh = plsc.VectorSubcoreMesh(core_axis_name="c", subcore_axis_name="s")

    @functools.partial(
        pl.kernel,
        mesh=mesh,
        out_type=jax.ShapeDtypeStruct((nseq, batch, hist, D), jnp.float32),
        scratch_types=[
            pltpu.VMEM((pairs_per_w * HP,), jnp.int32),
            pltpu.VMEM((NBUF, hist, D), jnp.float32),
        ]
        + [pltpu.SemaphoreType.DMA] * NBUF
        + [pltpu.SemaphoreType.DMA] * NBUF,
    )
    def gather_kernel(idx_hbm, table_hbm, out_hbm, idx_v, rows_v, *sems):
        gsem = sems[:NBUF]
        ssem = sems[NBUF:]
        wid = lax.axis_index("s") * info.num_cores + lax.axis_index("c")
        seq = wid // w_per_seq
        b0 = (wid % w_per_seq) * pairs_per_w
        base = pl.multiple_of(wid * pairs_per_w * HP, 8)
        pltpu.sync_copy(idx_hbm.at[pl.ds(base, pairs_per_w * HP)], idx_v)

        def gather_start(p, b):
            off = pl.multiple_of(p * HP, 8)
            pltpu.async_copy(
                table_hbm.at[idx_v.at[pl.ds(off, hist)]], rows_v.at[b], gsem[b]
            )

        def gather_wait(p, b):
            off = pl.multiple_of(p * HP, 8)
            pltpu.make_async_copy(
                table_hbm.at[idx_v.at[pl.ds(off, hist)]], rows_v.at[b], gsem[b]
            ).wait()

        def store_start(p, b):
            pltpu.async_copy(rows_v.at[b], out_hbm.at[seq, b0 + p], ssem[b])

        def store_wait(p, b):
            pltpu.make_async_copy(
                rows_v.at[b], out_hbm.at[seq, b0 + p], ssem[b]
            ).wait()

        for b in range(NBUF):
            gather_start(b, b)
        for b in range(NBUF):
            gather_wait(b, b)
            store_start(b, b)

        def body(st, carry):
            p0 = st * NBUF
            for b in range(NBUF):
                store_wait(p0 + b - NBUF, b)
                gather_start(p0 + b, b)
            for b in range(NBUF):
                gather_wait(p0 + b, b)
                store_start(p0 + b, b)
            return carry

        lax.fori_loop(1, nsteps, body, 0)

        for b in range(NBUF):
            store_wait((nsteps - 1) * NBUF + b, b)

    return gather_kernel


def kernel(a, b, embedding_table):
    batch, hist = a.shape
    idx = jnp.stack([a, b]).astype(jnp.int32)
    idx = jnp.pad(idx, ((0, 0), (0, 0), (0, HP - hist)))
    return _make_gather(2, batch, hist)(idx.reshape(-1), embedding_table)


# speedup vs baseline: 12.1230x; 1.8989x over previous
import functools

import jax
import jax.numpy as jnp
from jax import lax
from jax.experimental import pallas as pl
from jax.experimental.pallas import tpu as pltpu
from jax.experimental.pallas import tpu_sc as plsc

D = 128
CHUNK = 128
NBUF = 4


@functools.lru_cache(maxsize=None)
def _make_gather(total_rows: int):
    info = plsc.get_sparse_core_info()
    nw = info.num_cores * info.num_subcores
    assert total_rows % (nw * CHUNK * NBUF) == 0
    rows_per_w = total_rows // nw
    ngroups = rows_per_w // CHUNK
    nsteps = ngroups // NBUF
    mesh = plsc.VectorSubcoreMesh(core_axis_name="c", subcore_axis_name="s")

    @functools.partial(
        pl.kernel,
        mesh=mesh,
        out_type=jax.ShapeDtypeStruct((total_rows, D), jnp.float32),
        scratch_types=[
            pltpu.VMEM((rows_per_w,), jnp.int32),
            pltpu.VMEM((NBUF, CHUNK, D), jnp.float32),
        ]
        + [pltpu.SemaphoreType.DMA] * NBUF
        + [pltpu.SemaphoreType.DMA] * NBUF,
    )
    def gather_kernel(idx_hbm, table_hbm, out_hbm, idx_v, rows_v, *sems):
        gsem = sems[:NBUF]
        ssem = sems[NBUF:]
        wid = lax.axis_index("s") * info.num_cores + lax.axis_index("c")
        base = pl.multiple_of(wid * rows_per_w, 8)
        pltpu.sync_copy(idx_hbm.at[pl.ds(base, rows_per_w)], idx_v)

        def gather_start(g, b):
            off = pl.multiple_of(g * CHUNK, 8)
            pltpu.async_copy(
                table_hbm.at[idx_v.at[pl.ds(off, CHUNK)]], rows_v.at[b], gsem[b]
            )

        def gather_wait(g, b):
            off = pl.multiple_of(g * CHUNK, 8)
            pltpu.make_async_copy(
                table_hbm.at[idx_v.at[pl.ds(off, CHUNK)]], rows_v.at[b], gsem[b]
            ).wait()

        def store_start(g, b):
            off = pl.multiple_of(g * CHUNK, 8)
            pltpu.async_copy(
                rows_v.at[b], out_hbm.at[pl.ds(base + off, CHUNK)], ssem[b]
            )

        def store_wait(g, b):
            off = pl.multiple_of(g * CHUNK, 8)
            pltpu.make_async_copy(
                rows_v.at[b], out_hbm.at[pl.ds(base + off, CHUNK)], ssem[b]
            ).wait()

        for b in range(NBUF):
            gather_start(b, b)
        for b in range(NBUF):
            gather_wait(b, b)
            store_start(b, b)

        def body(p, carry):
            g0 = p * NBUF
            for b in range(NBUF):
                store_wait(g0 + b - NBUF, b)
                gather_start(g0 + b, b)
            for b in range(NBUF):
                gather_wait(g0 + b, b)
                store_start(g0 + b, b)
            return carry

        lax.fori_loop(1, nsteps, body, 0)

        gl = (nsteps - 1) * NBUF
        for b in range(NBUF):
            store_wait(gl + b, b)

    return gather_kernel


def kernel(a, b, embedding_table):
    batch, hist = a.shape
    idx = jnp.stack([a.T.astype(jnp.int32), b.T.astype(jnp.int32)])
    flat = _make_gather(2 * batch * hist)(idx.reshape(-1), embedding_table)
    return flat.reshape(2, hist, batch, D).transpose(0, 2, 1, 3)


# speedup vs baseline: 12.2955x; 1.0142x over previous
import functools

import jax
import jax.numpy as jnp
from jax import lax
from jax.experimental import pallas as pl
from jax.experimental.pallas import tpu as pltpu
from jax.experimental.pallas import tpu_sc as plsc

D = 128
CHUNK = 64
NBUF = 8


@functools.lru_cache(maxsize=None)
def _make_gather(total_rows: int):
    info = plsc.get_sparse_core_info()
    nw = info.num_cores * info.num_subcores
    assert total_rows % (nw * CHUNK * NBUF) == 0
    rows_per_w = total_rows // nw
    ngroups = rows_per_w // CHUNK
    nsteps = ngroups // NBUF
    mesh = plsc.VectorSubcoreMesh(core_axis_name="c", subcore_axis_name="s")

    @functools.partial(
        pl.kernel,
        mesh=mesh,
        out_type=jax.ShapeDtypeStruct((total_rows, D), jnp.float32),
        scratch_types=[
            pltpu.VMEM((rows_per_w,), jnp.int32),
            pltpu.VMEM((NBUF, CHUNK, D), jnp.float32),
        ]
        + [pltpu.SemaphoreType.DMA] * NBUF
        + [pltpu.SemaphoreType.DMA] * NBUF,
    )
    def gather_kernel(idx_hbm, table_hbm, out_hbm, idx_v, rows_v, *sems):
        gsem = sems[:NBUF]
        ssem = sems[NBUF:]
        wid = lax.axis_index("s") * info.num_cores + lax.axis_index("c")
        base = pl.multiple_of(wid * rows_per_w, 8)
        pltpu.sync_copy(idx_hbm.at[pl.ds(base, rows_per_w)], idx_v)

        def gather_start(g, b):
            off = pl.multiple_of(g * CHUNK, 8)
            pltpu.async_copy(
                table_hbm.at[idx_v.at[pl.ds(off, CHUNK)]], rows_v.at[b], gsem[b]
            )

        def gather_wait(g, b):
            off = pl.multiple_of(g * CHUNK, 8)
            pltpu.make_async_copy(
                table_hbm.at[idx_v.at[pl.ds(off, CHUNK)]], rows_v.at[b], gsem[b]
            ).wait()

        def store_start(g, b):
            off = pl.multiple_of(g * CHUNK, 8)
            pltpu.async_copy(
                rows_v.at[b], out_hbm.at[pl.ds(base + off, CHUNK)], ssem[b]
            )

        def store_wait(g, b):
            off = pl.multiple_of(g * CHUNK, 8)
            pltpu.make_async_copy(
                rows_v.at[b], out_hbm.at[pl.ds(base + off, CHUNK)], ssem[b]
            ).wait()

        for b in range(NBUF):
            gather_start(b, b)
        for b in range(NBUF):
            gather_wait(b, b)
            store_start(b, b)

        def body(p, carry):
            g0 = p * NBUF
            for b in range(NBUF):
                store_wait(g0 + b - NBUF, b)
                gather_start(g0 + b, b)
            for b in range(NBUF):
                gather_wait(g0 + b, b)
                store_start(g0 + b, b)
            return carry

        lax.fori_loop(1, nsteps, body, 0)

        gl = (nsteps - 1) * NBUF
        for b in range(NBUF):
            store_wait(gl + b, b)

    return gather_kernel


def kernel(a, b, embedding_table):
    batch, hist = a.shape
    idx = jnp.stack([a.T.astype(jnp.int32), b.T.astype(jnp.int32)])
    flat = _make_gather(2 * batch * hist)(idx.reshape(-1), embedding_table)
    return flat.reshape(2, hist, batch, D).transpose(0, 2, 1, 3)
